# Initial kernel scaffold; baseline (speedup 1.0000x reference)
import jax
import jax.numpy as jnp
from jax.experimental import pallas as pl


def kernel(x, table, bias):
    raise NotImplementedError("write your pallas kernel here")


# speedup vs baseline: 1.0312x; 1.0312x over previous
import functools

import jax
import jax.numpy as jnp
from jax import lax
from jax.experimental import pallas as pl
from jax.experimental.pallas import tpu as pltpu
from jax.experimental.pallas import tpu_sc as plsc

BATCH = 16384
FIELDS = 100
PADF = 101
NC = 2
NS = 16
NW = NC * NS
ROWS_W = BATCH // NW
CHUNK = ROWS_W * PADF
LANES = 16


def _body(x_hbm, tab_hbm, bias_hbm, out_hbm, idx_v, vals_v, out_v, bias_v,
          sem):
    wid = lax.axis_index("s") * NC + lax.axis_index("c")
    base = wid * CHUNK

    pltpu.sync_copy(bias_hbm, bias_v)
    pltpu.sync_copy(x_hbm.at[pl.ds(base, CHUNK)], idx_v)
    pltpu.async_copy(tab_hbm.at[idx_v], vals_v, sem).wait()

    bias_vec = bias_v[...]
    lane_off = lax.iota(jnp.int32, LANES) * PADF

    def outer(i, carry):
        row0 = i * LANES
        ibase = lane_off + row0 * PADF

        def inner(j, acc):
            return acc + plsc.load_gather(vals_v, [ibase + j])

        acc = lax.fori_loop(0, FIELDS, inner,
                            jnp.zeros((LANES,), jnp.float32), unroll=4)
        z = acc + bias_vec
        out_v[pl.ds(row0, LANES)] = 1.0 / (1.0 + jnp.exp(-z))
        return carry

    lax.fori_loop(0, ROWS_W // LANES, outer, 0)
    pltpu.sync_copy(out_v, out_hbm.at[pl.ds(wid * ROWS_W, ROWS_W)])


@jax.jit
def _run(x_pad_flat, table_flat, bias16):
    mesh = plsc.VectorSubcoreMesh(core_axis_name="c", subcore_axis_name="s")
    f = pl.kernel(
        _body,
        out_type=jax.ShapeDtypeStruct((BATCH,), jnp.float32),
        mesh=mesh,
        scratch_types=[
            pltpu.VMEM((CHUNK,), jnp.int32),
            pltpu.VMEM((CHUNK,), jnp.float32),
            pltpu.VMEM((ROWS_W,), jnp.float32),
            pltpu.VMEM((LANES,), jnp.float32),
            pltpu.SemaphoreType.DMA,
        ],
        compiler_params=pltpu.CompilerParams(needs_layout_passes=False),
    )
    return f(x_pad_flat, table_flat, bias16)


def kernel(x, table, bias):
    x_pad = jnp.pad(x, ((0, 0), (0, PADF - FIELDS)))
    x_flat = x_pad.reshape(-1)
    table_flat = table.reshape(-1)
    bias16 = jnp.broadcast_to(bias, (LANES,))
    return _run(x_flat, table_flat, bias16)


# speedup vs baseline: 1.3242x; 1.2841x over previous
import jax
import jax.numpy as jnp
from jax import lax
from jax.experimental import pallas as pl
from jax.experimental.pallas import tpu as pltpu
from jax.experimental.pallas import tpu_sc as plsc

BATCH = 16384
FIELDS = 100
NC = 2
NS = 16
NW = NC * NS
ROWS_W = BATCH // NW
CHUNK = ROWS_W * FIELDS
LANES = 16


def _body(x_hbm, tab_hbm, bias_hbm, out_hbm, idx_v, vals_v, out_v, bias_v,
          sem):
    wid = lax.axis_index("s") * NC + lax.axis_index("c")
    base = wid * CHUNK

    pltpu.sync_copy(bias_hbm, bias_v)
    pltpu.sync_copy(x_hbm.at[pl.ds(base, CHUNK)], idx_v)
    pltpu.async_copy(tab_hbm.at[idx_v], vals_v, sem).wait()

    bias_vec = bias_v[...]
    lane_off = lax.iota(jnp.int32, LANES) * FIELDS

    def outer(i, carry):
        row0 = i * LANES
        ibase = lane_off + row0 * FIELDS

        def inner(j, acc):
            return acc + plsc.load_gather(vals_v, [ibase + j])

        acc = lax.fori_loop(0, FIELDS, inner,
                            jnp.zeros((LANES,), jnp.float32), unroll=4)
        z = acc + bias_vec
        out_v[pl.ds(row0, LANES)] = 1.0 / (1.0 + jnp.exp(-z))
        return carry

    lax.fori_loop(0, ROWS_W // LANES, outer, 0)
    pltpu.sync_copy(out_v, out_hbm.at[pl.ds(wid * ROWS_W, ROWS_W)])


@jax.jit
def _run(x_flat, table_flat, bias):
    mesh = plsc.VectorSubcoreMesh(core_axis_name="c", subcore_axis_name="s")
    f = pl.kernel(
        _body,
        out_type=jax.ShapeDtypeStruct((BATCH,), jnp.float32),
        mesh=mesh,
        scratch_types=[
            pltpu.VMEM((CHUNK,), jnp.int32),
            pltpu.VMEM((CHUNK,), jnp.float32),
            pltpu.VMEM((ROWS_W,), jnp.float32),
            pltpu.VMEM((LANES,), jnp.float32),
            pltpu.SemaphoreType.DMA,
        ],
        compiler_params=pltpu.CompilerParams(needs_layout_passes=False),
    )
    return f(x_flat, table_flat, bias)


def kernel(x, table, bias):
    return _run(x.reshape(-1), table.reshape(-1),
                jnp.broadcast_to(bias, (LANES,)))
